# Initial kernel scaffold; baseline (speedup 1.0000x reference)
import jax
import jax.numpy as jnp
from jax.experimental import pallas as pl


def kernel(points_reco, points_gen, features_reco, features_gen, W1, b1, W2, b2):
    raise NotImplementedError("write your pallas kernel here")


# speedup vs baseline: 12.5445x; 12.5445x over previous
import functools
import math

import jax
import jax.numpy as jnp
from jax import lax
from jax.experimental import pallas as pl
from jax.experimental.pallas import tpu as pltpu

K = 10
B, N, PD = 64, 1024, 96
BI = 256


def _gelu(x):
    return 0.5 * x * (1.0 + lax.erf(x * (1.0 / math.sqrt(2.0))))


def _pet_kernel(xr_ref, yr_ref, xg_ref, yg_ref, fr_ref, fg_ref,
                w1a_ref, w1d_ref, b1_ref, w2_ref, b2_ref, out_ref):
    xi = xr_ref[0]
    yi = yr_ref[0]
    xj = xg_ref[0]
    yj = yg_ref[0]
    r_a = xi * xi + yi * yi
    r_b = xj * xj + yj * yj
    xib = xi.astype(jnp.bfloat16).astype(jnp.float32)
    yib = yi.astype(jnp.bfloat16).astype(jnp.float32)
    xjb = xj.astype(jnp.bfloat16).astype(jnp.float32)
    yjb = yj.astype(jnp.bfloat16).astype(jnp.float32)
    m = xib * xjb + yib * yjb
    d = r_a - 2.0 * m + r_b + 1e-05

    fg = fg_ref[0]
    fr = fr_ref[0]
    c = jnp.dot(fr, w1d_ref[...], preferred_element_type=jnp.float32)
    c = c + b1_ref[...]

    iota = lax.broadcasted_iota(jnp.int32, (BI, N), 1)
    acc = jnp.full((BI, PD), -jnp.inf, dtype=jnp.float32)
    for _ in range(K):
        v = jnp.min(d, axis=1, keepdims=True)
        t = jnp.where(d == v, iota, N)
        idx = jnp.min(t, axis=1, keepdims=True)
        sel = iota == idx
        oh = sel.astype(jnp.float32)
        d = jnp.where(sel, jnp.inf, d)
        fk = jnp.dot(oh, fg, preferred_element_type=jnp.float32)
        h = jnp.dot(fk, w1a_ref[...], preferred_element_type=jnp.float32) + c
        h = _gelu(h)
        h2 = jnp.dot(h, w2_ref[...], preferred_element_type=jnp.float32)
        h2 = _gelu(h2 + b2_ref[...])
        acc = jnp.maximum(acc, h2)
    out_ref[0] = acc


@jax.jit
def kernel(points_reco, points_gen, features_reco, features_gen, W1, b1, W2, b2):
    xr = points_reco[..., 0:1]
    yr = points_reco[..., 1:2]
    xg = points_gen[..., 0].reshape(B, 1, N)
    yg = points_gen[..., 1].reshape(B, 1, N)
    w1a = W1[:PD]
    w1d = W1[PD:] - W1[:PD]
    b1r = b1.reshape(1, 4 * PD)
    b2r = b2.reshape(1, PD)

    grid = (B, N // BI)
    out = pl.pallas_call(
        _pet_kernel,
        grid=grid,
        in_specs=[
            pl.BlockSpec((1, BI, 1), lambda b, i: (b, i, 0)),
            pl.BlockSpec((1, BI, 1), lambda b, i: (b, i, 0)),
            pl.BlockSpec((1, 1, N), lambda b, i: (b, 0, 0)),
            pl.BlockSpec((1, 1, N), lambda b, i: (b, 0, 0)),
            pl.BlockSpec((1, BI, PD), lambda b, i: (b, i, 0)),
            pl.BlockSpec((1, N, PD), lambda b, i: (b, 0, 0)),
            pl.BlockSpec((PD, 4 * PD), lambda b, i: (0, 0)),
            pl.BlockSpec((PD, 4 * PD), lambda b, i: (0, 0)),
            pl.BlockSpec((1, 4 * PD), lambda b, i: (0, 0)),
            pl.BlockSpec((4 * PD, PD), lambda b, i: (0, 0)),
            pl.BlockSpec((1, PD), lambda b, i: (0, 0)),
        ],
        out_specs=pl.BlockSpec((1, BI, PD), lambda b, i: (b, i, 0)),
        out_shape=jax.ShapeDtypeStruct((B, N, PD), jnp.float32),
        compiler_params=pltpu.CompilerParams(
            dimension_semantics=("parallel", "parallel"),
        ),
    )(xr, yr, xg, yg, features_reco, features_gen, w1a, w1d, b1r, W2, b2r)
    return out


# speedup vs baseline: 17.1727x; 1.3689x over previous
import functools
import math

import jax
import jax.numpy as jnp
from jax import lax
from jax.experimental import pallas as pl
from jax.experimental.pallas import tpu as pltpu

K = 10
B, N, PD = 64, 1024, 96
BI = 256


def _gelu(x):
    return 0.5 * x * (1.0 + lax.erf(x * (1.0 / math.sqrt(2.0))))


def _pet_kernel(xr_ref, yr_ref, xg_ref, yg_ref, fr_ref, fg_ref,
                w1a_ref, w1d_ref, b1_ref, w2_ref, b2_ref, out_ref):
    xi = xr_ref[0]
    yi = yr_ref[0]
    xj = xg_ref[0]
    yj = yg_ref[0]
    r_a = xi * xi + yi * yi
    r_b = xj * xj + yj * yj
    xib = xi.astype(jnp.bfloat16).astype(jnp.float32)
    yib = yi.astype(jnp.bfloat16).astype(jnp.float32)
    xjb = xj.astype(jnp.bfloat16).astype(jnp.float32)
    yjb = yj.astype(jnp.bfloat16).astype(jnp.float32)
    m = xib * xjb + yib * yjb
    d = r_a - 2.0 * m + r_b + 1e-05

    fg = fg_ref[0]
    fr = fr_ref[0]
    c = jnp.dot(fr, w1d_ref[...], preferred_element_type=jnp.float32)
    c = c + b1_ref[...]

    fgb = fg.astype(jnp.bfloat16)
    acc = jnp.full((BI, PD), -jnp.inf, dtype=jnp.float32)
    for _ in range(K):
        v = jnp.min(d, axis=1, keepdims=True)
        sel = d == v
        oh = sel.astype(jnp.bfloat16)
        d = jnp.where(sel, jnp.inf, d)
        fk = jnp.dot(oh, fgb, preferred_element_type=jnp.float32)
        h = jnp.dot(fk, w1a_ref[...], preferred_element_type=jnp.float32) + c
        h = _gelu(h)
        h2 = jnp.dot(h, w2_ref[...], preferred_element_type=jnp.float32)
        h2 = _gelu(h2 + b2_ref[...])
        acc = jnp.maximum(acc, h2)
    out_ref[0] = acc


@jax.jit
def kernel(points_reco, points_gen, features_reco, features_gen, W1, b1, W2, b2):
    xr = points_reco[..., 0:1]
    yr = points_reco[..., 1:2]
    xg = points_gen[..., 0].reshape(B, 1, N)
    yg = points_gen[..., 1].reshape(B, 1, N)
    w1a = W1[:PD]
    w1d = W1[PD:] - W1[:PD]
    b1r = b1.reshape(1, 4 * PD)
    b2r = b2.reshape(1, PD)

    grid = (B, N // BI)
    out = pl.pallas_call(
        _pet_kernel,
        grid=grid,
        in_specs=[
            pl.BlockSpec((1, BI, 1), lambda b, i: (b, i, 0)),
            pl.BlockSpec((1, BI, 1), lambda b, i: (b, i, 0)),
            pl.BlockSpec((1, 1, N), lambda b, i: (b, 0, 0)),
            pl.BlockSpec((1, 1, N), lambda b, i: (b, 0, 0)),
            pl.BlockSpec((1, BI, PD), lambda b, i: (b, i, 0)),
            pl.BlockSpec((1, N, PD), lambda b, i: (b, 0, 0)),
            pl.BlockSpec((PD, 4 * PD), lambda b, i: (0, 0)),
            pl.BlockSpec((PD, 4 * PD), lambda b, i: (0, 0)),
            pl.BlockSpec((1, 4 * PD), lambda b, i: (0, 0)),
            pl.BlockSpec((4 * PD, PD), lambda b, i: (0, 0)),
            pl.BlockSpec((1, PD), lambda b, i: (0, 0)),
        ],
        out_specs=pl.BlockSpec((1, BI, PD), lambda b, i: (b, i, 0)),
        out_shape=jax.ShapeDtypeStruct((B, N, PD), jnp.float32),
        compiler_params=pltpu.CompilerParams(
            dimension_semantics=("parallel", "parallel"),
        ),
    )(xr, yr, xg, yg, features_reco, features_gen, w1a, w1d, b1r, W2, b2r)
    return out
